# baseline (speedup 1.0000x reference)
import functools

import jax
import jax.numpy as jnp
from jax import lax
from jax.experimental import pallas as pl
from jax.experimental.pallas import tpu as pltpu
from jax.experimental.pallas import tpu_sc as plsc

_NUM_STEPS = 100
_MARGIN = 0.1
_TILE = 512
_NCODES = 256


def _stats_body(emb_ref, labc_ref, norm_ref, stats_ref):
    e = emb_ref[...]
    n = e.shape[0]
    scale = 1.0 / jnp.maximum(jnp.sqrt(jnp.sum(e * e, axis=1, keepdims=True)),
                              1e-12)
    en = e * scale
    norm_ref[...] = en

    cls = lax.broadcasted_iota(jnp.int32, (128, n), 0)
    oh = (labc_ref[...] == cls).astype(jnp.float32)
    class_sums = lax.dot_general(oh, en, (((1,), (0,)), ((), ())),
                                 preferred_element_type=jnp.float32)
    counts = jnp.sum(oh, axis=1)

    s_eq = jnp.sum(class_sums * class_sums)
    n_eq = jnp.sum(counts * counts)
    colsum = jnp.sum(class_sums, axis=0)
    s_all = jnp.sum(colsum * colsum)
    trace = jnp.sum(en * en)

    row = lax.broadcasted_iota(jnp.int32, (8, 128), 0)
    lane = lax.broadcasted_iota(jnp.int32, (8, 128), 1)
    on_r0 = row == 0
    stats_ref[...] = (jnp.where(on_r0 & (lane == 0), s_all, 0.0)
                      + jnp.where(on_r0 & (lane == 1), s_eq, 0.0)
                      + jnp.where(on_r0 & (lane == 2), n_eq, 0.0)
                      + jnp.where(on_r0 & (lane == 3), trace, 0.0))


def _tc_stats(emb, labels):
    b, d = emb.shape
    return pl.pallas_call(
        _stats_body,
        out_shape=[
            jax.ShapeDtypeStruct((b, d), jnp.float32),
            jax.ShapeDtypeStruct((8, 128), jnp.float32),
        ],
    )(emb, labels.reshape(1, b))


def _codes_body(base_tile, enr_ref, enc_ref, labr_ref, labc_ref, codes_ref):
    bi = pl.program_id(0) + base_tile
    bj = pl.program_id(1)
    t = _TILE

    sim = lax.dot_general(enr_ref[...], enc_ref[...], (((1,), (1,)), ((), ())),
                          preferred_element_type=jnp.float32)

    eq = labr_ref[...] == labc_ref[...]
    bin_idx = jnp.clip(((sim + 1.0) / 2.0 * _NUM_STEPS).astype(jnp.int32),
                       0, _NUM_STEPS - 1)
    code = jnp.where(eq, bin_idx + _NUM_STEPS, bin_idx)

    def pack2(c):
        h = t // 2
        return (c[:h, :] | (c[h:, :] << 16)).reshape(codes_ref.shape)

    @pl.when(bi == bj)
    def _():
        ld = (lax.broadcasted_iota(jnp.int32, (t, t), 0)
              == lax.broadcasted_iota(jnp.int32, (t, t), 1))
        codes_ref[...] = pack2(jnp.where(ld, _NCODES - 1, code))

    @pl.when(bi != bj)
    def _():
        codes_ref[...] = pack2(code)


def _tc_codes_band(en, labels, band, nb):
    b, d = en.shape
    nt = b // _TILE
    bt = nt // nb
    base = band * bt
    rows_per_blk = _TILE * _TILE // 256
    labr = labels.reshape(b, 1)
    labc = labels.reshape(1, b)
    return pl.pallas_call(
        functools.partial(_codes_body, base),
        grid=(bt, nt),
        in_specs=[
            pl.BlockSpec((_TILE, d), lambda i, j, base=base: (base + i, 0)),
            pl.BlockSpec((_TILE, d), lambda i, j: (j, 0)),
            pl.BlockSpec((_TILE, 1), lambda i, j, base=base: (base + i, 0)),
            pl.BlockSpec((1, _TILE), lambda i, j: (0, j)),
        ],
        out_specs=pl.BlockSpec((rows_per_blk, 128),
                               lambda i, j, nt=nt: (i * nt + j, 0)),
        out_shape=jax.ShapeDtypeStruct((b * b // 256 // nb, 128), jnp.int32),
    )(en, en, labr, labc)


def _make_sc_hist(total):
    info = plsc.get_sparse_core_info()
    nc, ns = info.num_cores, info.num_subcores
    nw = nc * ns
    words_per_w = total // nw
    ch = 32768
    nchunk = words_per_w // ch
    hist_words = 16 * _NCODES
    mesh = plsc.VectorSubcoreMesh(core_axis_name="c", subcore_axis_name="s")

    @functools.partial(
        pl.kernel, mesh=mesh,
        out_type=jax.ShapeDtypeStruct((nw * hist_words,), jnp.float32),
        compiler_params=pltpu.CompilerParams(needs_layout_passes=False),
        scratch_types=[
            pltpu.VMEM((2, ch), jnp.int32),
            pltpu.VMEM((hist_words,), jnp.float32),
            pltpu.SemaphoreType.DMA,
            pltpu.SemaphoreType.DMA,
        ],
    )
    def sc_hist(codes_hbm, out_hbm, buf, hist, sem0, sem1):
        wid = lax.axis_index("s") * nc + lax.axis_index("c")
        base = wid * words_per_w
        sems = (sem0, sem1)

        zero16 = jnp.zeros((16,), jnp.float32)

        def zinit(i, _):
            hist[pl.ds(i * 16, 16)] = zero16
            return 0

        lax.fori_loop(0, hist_words // 16, zinit, 0)

        ones16 = jnp.full((16,), 1.0, jnp.float32)
        laneoff = lax.iota(jnp.int32, 16)

        copies = []
        copies.append(pltpu.async_copy(
            codes_hbm.at[pl.ds(base, ch)], buf.at[0], sems[0]))

        for g in range(nchunk):
            bsel = g % 2
            if g + 1 < nchunk:
                copies.append(pltpu.async_copy(
                    codes_hbm.at[pl.ds(base + (g + 1) * ch, ch)],
                    buf.at[(g + 1) % 2], sems[(g + 1) % 2]))
            copies[g].wait()

            @plsc.parallel_loop(0, ch, step=128, unroll=4)
            def _(k, bsel=bsel):
                for u in range(8):
                    w = buf[bsel, pl.ds(k + u * 16, 16)]
                    lo = w & 0xFFFF
                    hi = lax.shift_right_logical(w, 16)
                    plsc.addupdate_scatter(hist, [lo * 16 + laneoff], ones16)
                    plsc.addupdate_scatter(hist, [hi * 16 + laneoff], ones16)

        pltpu.sync_copy(hist, out_hbm.at[pl.ds(wid * hist_words, hist_words)])

    return sc_hist, nw


def kernel(embeddings, labels):
    b = embeddings.shape[0]
    labels = labels.astype(jnp.int32)

    en, stats = _tc_stats(embeddings, labels)

    nb = 4
    sc_hist, nw = _make_sc_hist(b * b // 2 // nb)
    partials = []
    for band in range(nb):
        codes = _tc_codes_band(en, labels, band, nb)
        partials.append(sc_hist(codes.reshape(-1)))
    hist = jnp.sum(jnp.stack(partials).reshape(nb * nw, _NCODES, 16),
                   axis=(0, 2))

    neg_hist = hist[:_NUM_STEPS]
    pos_hist = hist[_NUM_STEPS:2 * _NUM_STEPS]
    pos_hist = pos_hist / (jnp.sum(pos_hist) + 1e-16)
    neg_hist = neg_hist / (jnp.sum(neg_hist) + 1e-16)
    overlap = jnp.sum(jnp.minimum(pos_hist, neg_hist))

    s_all = stats[0, 0]
    s_eq = stats[0, 1]
    n_eq = stats[0, 2]
    trace = stats[0, 3]
    bf = jnp.float32(b)
    pos_mean = (s_eq - trace) / (n_eq - bf)
    neg_mean = (s_all - s_eq) / (bf * bf - n_eq)

    return overlap + jax.nn.relu(_MARGIN - (pos_mean - neg_mean))


# speedup vs baseline: 1.0183x; 1.0183x over previous
import functools

import jax
import jax.numpy as jnp
from jax import lax
from jax.experimental import pallas as pl
from jax.experimental.pallas import tpu as pltpu
from jax.experimental.pallas import tpu_sc as plsc

_NUM_STEPS = 100
_MARGIN = 0.1
_TILE = 512
_NCODES = 256


def _stats_body(emb_ref, labc_ref, norm_ref, stats_ref):
    e = emb_ref[...]
    n = e.shape[0]
    scale = 1.0 / jnp.maximum(jnp.sqrt(jnp.sum(e * e, axis=1, keepdims=True)),
                              1e-12)
    en = e * scale
    norm_ref[...] = en

    cls = lax.broadcasted_iota(jnp.int32, (128, n), 0)
    oh = (labc_ref[...] == cls).astype(jnp.float32)
    class_sums = lax.dot_general(oh, en, (((1,), (0,)), ((), ())),
                                 preferred_element_type=jnp.float32)
    counts = jnp.sum(oh, axis=1)

    s_eq = jnp.sum(class_sums * class_sums)
    n_eq = jnp.sum(counts * counts)
    colsum = jnp.sum(class_sums, axis=0)
    s_all = jnp.sum(colsum * colsum)
    trace = jnp.sum(en * en)

    row = lax.broadcasted_iota(jnp.int32, (8, 128), 0)
    lane = lax.broadcasted_iota(jnp.int32, (8, 128), 1)
    on_r0 = row == 0
    stats_ref[...] = (jnp.where(on_r0 & (lane == 0), s_all, 0.0)
                      + jnp.where(on_r0 & (lane == 1), s_eq, 0.0)
                      + jnp.where(on_r0 & (lane == 2), n_eq, 0.0)
                      + jnp.where(on_r0 & (lane == 3), trace, 0.0))


def _tc_stats(emb, labels):
    b, d = emb.shape
    return pl.pallas_call(
        _stats_body,
        out_shape=[
            jax.ShapeDtypeStruct((b, d), jnp.float32),
            jax.ShapeDtypeStruct((8, 128), jnp.float32),
        ],
    )(emb, labels.reshape(1, b))


def _codes_body(base_tile, enr_ref, enc_ref, labr_ref, labc_ref, codes_ref):
    bi = pl.program_id(0) + base_tile
    bj = pl.program_id(1)
    t = _TILE

    sim = lax.dot_general(enr_ref[...], enc_ref[...], (((1,), (1,)), ((), ())),
                          preferred_element_type=jnp.float32)

    eq = labr_ref[...] == labc_ref[...]
    bin_idx = jnp.clip(((sim + 1.0) / 2.0 * _NUM_STEPS).astype(jnp.int32),
                       0, _NUM_STEPS - 1)
    code = jnp.where(eq, bin_idx + _NUM_STEPS, bin_idx)

    def pack2(c):
        h = t // 2
        return (c[:h, :] | (c[h:, :] << 16)).reshape(codes_ref.shape)

    @pl.when(bi == bj)
    def _():
        ld = (lax.broadcasted_iota(jnp.int32, (t, t), 0)
              == lax.broadcasted_iota(jnp.int32, (t, t), 1))
        codes_ref[...] = pack2(jnp.where(ld, _NCODES - 1, code))

    @pl.when(bi != bj)
    def _():
        codes_ref[...] = pack2(code)


def _tc_codes_band(en, labels, band, nb):
    b, d = en.shape
    nt = b // _TILE
    bt = nt // nb
    base = band * bt
    rows_per_blk = _TILE * _TILE // 256
    labr = labels.reshape(b, 1)
    labc = labels.reshape(1, b)
    return pl.pallas_call(
        functools.partial(_codes_body, base),
        grid=(bt, nt),
        in_specs=[
            pl.BlockSpec((_TILE, d), lambda i, j, base=base: (base + i, 0)),
            pl.BlockSpec((_TILE, d), lambda i, j: (j, 0)),
            pl.BlockSpec((_TILE, 1), lambda i, j, base=base: (base + i, 0)),
            pl.BlockSpec((1, _TILE), lambda i, j: (0, j)),
        ],
        out_specs=pl.BlockSpec((rows_per_blk, 128),
                               lambda i, j, nt=nt: (i * nt + j, 0)),
        out_shape=jax.ShapeDtypeStruct((b * b // 256 // nb, 128), jnp.int32),
    )(en, en, labr, labc)


def _make_sc_hist(total):
    info = plsc.get_sparse_core_info()
    nc, ns = info.num_cores, info.num_subcores
    nw = nc * ns
    words_per_w = total // nw
    ch = 32768
    nchunk = words_per_w // ch
    hist_words = 16 * _NCODES
    mesh = plsc.VectorSubcoreMesh(core_axis_name="c", subcore_axis_name="s")

    @functools.partial(
        pl.kernel, mesh=mesh,
        out_type=jax.ShapeDtypeStruct((nw * hist_words,), jnp.float32),
        compiler_params=pltpu.CompilerParams(needs_layout_passes=False),
        scratch_types=[
            pltpu.VMEM((2, ch), jnp.int32),
            pltpu.VMEM((hist_words,), jnp.float32),
            pltpu.SemaphoreType.DMA,
            pltpu.SemaphoreType.DMA,
        ],
    )
    def sc_hist(codes_hbm, out_hbm, buf, hist, sem0, sem1):
        wid = lax.axis_index("s") * nc + lax.axis_index("c")
        base = wid * words_per_w
        sems = (sem0, sem1)

        zero16 = jnp.zeros((16,), jnp.float32)

        def zinit(i, _):
            hist[pl.ds(i * 16, 16)] = zero16
            return 0

        lax.fori_loop(0, hist_words // 16, zinit, 0)

        ones16 = jnp.full((16,), 1.0, jnp.float32)
        laneoff = lax.iota(jnp.int32, 16)

        copies = []
        copies.append(pltpu.async_copy(
            codes_hbm.at[pl.ds(base, ch)], buf.at[0], sems[0]))

        for g in range(nchunk):
            bsel = g % 2
            if g + 1 < nchunk:
                copies.append(pltpu.async_copy(
                    codes_hbm.at[pl.ds(base + (g + 1) * ch, ch)],
                    buf.at[(g + 1) % 2], sems[(g + 1) % 2]))
            copies[g].wait()

            @plsc.parallel_loop(0, ch, step=128, unroll=4)
            def _(k, bsel=bsel):
                for u in range(8):
                    w = buf[bsel, pl.ds(k + u * 16, 16)]
                    lo = w & 0xFFFF
                    hi = lax.shift_right_logical(w, 16)
                    plsc.addupdate_scatter(hist, [lo * 16 + laneoff], ones16)
                    plsc.addupdate_scatter(hist, [hi * 16 + laneoff], ones16)

        pltpu.sync_copy(hist, out_hbm.at[pl.ds(wid * hist_words, hist_words)])

    return sc_hist, nw


def kernel(embeddings, labels):
    b = embeddings.shape[0]
    labels = labels.astype(jnp.int32)

    en, stats = _tc_stats(embeddings, labels)

    nb = 2
    sc_hist, nw = _make_sc_hist(b * b // 2 // nb)
    partials = []
    for band in range(nb):
        codes = _tc_codes_band(en, labels, band, nb)
        partials.append(sc_hist(codes.reshape(-1)))
    hist = jnp.sum(jnp.stack(partials).reshape(nb * nw, _NCODES, 16),
                   axis=(0, 2))

    neg_hist = hist[:_NUM_STEPS]
    pos_hist = hist[_NUM_STEPS:2 * _NUM_STEPS]
    pos_hist = pos_hist / (jnp.sum(pos_hist) + 1e-16)
    neg_hist = neg_hist / (jnp.sum(neg_hist) + 1e-16)
    overlap = jnp.sum(jnp.minimum(pos_hist, neg_hist))

    s_all = stats[0, 0]
    s_eq = stats[0, 1]
    n_eq = stats[0, 2]
    trace = stats[0, 3]
    bf = jnp.float32(b)
    pos_mean = (s_eq - trace) / (n_eq - bf)
    neg_mean = (s_all - s_eq) / (bf * bf - n_eq)

    return overlap + jax.nn.relu(_MARGIN - (pos_mean - neg_mean))


# speedup vs baseline: 1.2255x; 1.2035x over previous
import functools

import jax
import jax.numpy as jnp
from jax import lax
from jax.experimental import pallas as pl
from jax.experimental.pallas import tpu as pltpu
from jax.experimental.pallas import tpu_sc as plsc

_NUM_STEPS = 100
_MARGIN = 0.1
_TILE = 512
_NCODES = 256


def _stats_body(emb_ref, labc_ref, norm_ref, stats_ref):
    e = emb_ref[...]
    n = e.shape[0]
    scale = 1.0 / jnp.maximum(jnp.sqrt(jnp.sum(e * e, axis=1, keepdims=True)),
                              1e-12)
    en = e * scale
    norm_ref[...] = en

    cls = lax.broadcasted_iota(jnp.int32, (128, n), 0)
    oh = (labc_ref[...] == cls).astype(jnp.float32)
    class_sums = lax.dot_general(oh, en, (((1,), (0,)), ((), ())),
                                 preferred_element_type=jnp.float32)
    counts = jnp.sum(oh, axis=1)

    s_eq = jnp.sum(class_sums * class_sums)
    n_eq = jnp.sum(counts * counts)
    colsum = jnp.sum(class_sums, axis=0)
    s_all = jnp.sum(colsum * colsum)
    trace = jnp.sum(en * en)

    row = lax.broadcasted_iota(jnp.int32, (8, 128), 0)
    lane = lax.broadcasted_iota(jnp.int32, (8, 128), 1)
    on_r0 = row == 0
    stats_ref[...] = (jnp.where(on_r0 & (lane == 0), s_all, 0.0)
                      + jnp.where(on_r0 & (lane == 1), s_eq, 0.0)
                      + jnp.where(on_r0 & (lane == 2), n_eq, 0.0)
                      + jnp.where(on_r0 & (lane == 3), trace, 0.0))


def _tc_stats(emb, labels):
    b, d = emb.shape
    return pl.pallas_call(
        _stats_body,
        out_shape=[
            jax.ShapeDtypeStruct((b, d), jnp.float32),
            jax.ShapeDtypeStruct((8, 128), jnp.float32),
        ],
    )(emb, labels.reshape(1, b))


def _codes_body(base_tile, enr_ref, enc_ref, labr_ref, labc_ref, codes_ref):
    bi = pl.program_id(0) + base_tile
    bj = pl.program_id(1)
    t = _TILE

    sim = lax.dot_general(enr_ref[...], enc_ref[...], (((1,), (1,)), ((), ())),
                          preferred_element_type=jnp.float32)

    eq = labr_ref[...] == labc_ref[...]
    bin_idx = jnp.minimum(((sim + 1.0) / 2.0 * _NUM_STEPS).astype(jnp.int32),
                          _NUM_STEPS - 1)
    code = jnp.where(eq, bin_idx + _NUM_STEPS, bin_idx)

    def pack4(c):
        q = t // 4
        w = (c[:q, :] | (c[q:2 * q, :] << 8)
             | (c[2 * q:3 * q, :] << 16) | (c[3 * q:, :] << 24))
        return w.reshape(codes_ref.shape)

    @pl.when(bi == bj)
    def _():
        ld = (lax.broadcasted_iota(jnp.int32, (t, t), 0)
              == lax.broadcasted_iota(jnp.int32, (t, t), 1))
        codes_ref[...] = pack4(jnp.where(ld, _NCODES - 1, code))

    @pl.when(bi != bj)
    def _():
        codes_ref[...] = pack4(code)


def _tc_codes_band(en, labels, base, bt):
    b, d = en.shape
    nt = b // _TILE
    rows_per_blk = _TILE * _TILE // 4 // 128
    labr = labels.reshape(b, 1)
    labc = labels.reshape(1, b)
    return pl.pallas_call(
        functools.partial(_codes_body, base),
        grid=(bt, nt),
        in_specs=[
            pl.BlockSpec((_TILE, d), lambda i, j, base=base: (base + i, 0)),
            pl.BlockSpec((_TILE, d), lambda i, j: (j, 0)),
            pl.BlockSpec((_TILE, 1), lambda i, j, base=base: (base + i, 0)),
            pl.BlockSpec((1, _TILE), lambda i, j: (0, j)),
        ],
        out_specs=pl.BlockSpec((rows_per_blk, 128),
                               lambda i, j, nt=nt: (i * nt + j, 0)),
        out_shape=jax.ShapeDtypeStruct((bt * nt * rows_per_blk, 128),
                                       jnp.int32),
    )(en, en, labr, labc)


def _make_sc_hist(total):
    info = plsc.get_sparse_core_info()
    nc, ns = info.num_cores, info.num_subcores
    nw = nc * ns
    words_per_w = total // nw
    ch = 32768
    nchunk = words_per_w // ch
    hist_words = 16 * _NCODES
    mesh = plsc.VectorSubcoreMesh(core_axis_name="c", subcore_axis_name="s")

    @functools.partial(
        pl.kernel, mesh=mesh,
        out_type=jax.ShapeDtypeStruct((nw * hist_words,), jnp.float32),
        compiler_params=pltpu.CompilerParams(needs_layout_passes=False),
        scratch_types=[
            pltpu.VMEM((2, ch), jnp.int32),
            pltpu.VMEM((hist_words,), jnp.float32),
            pltpu.SemaphoreType.DMA,
            pltpu.SemaphoreType.DMA,
        ],
    )
    def sc_hist(codes_hbm, out_hbm, buf, hist, sem0, sem1):
        wid = lax.axis_index("s") * nc + lax.axis_index("c")
        base = wid * words_per_w
        sems = (sem0, sem1)

        zero16 = jnp.zeros((16,), jnp.float32)

        def zinit(i, _):
            hist[pl.ds(i * 16, 16)] = zero16
            return 0

        lax.fori_loop(0, hist_words // 16, zinit, 0)

        ones16 = jnp.full((16,), 1.0, jnp.float32)
        laneoff = lax.iota(jnp.int32, 16)

        copies = []
        copies.append(pltpu.async_copy(
            codes_hbm.at[pl.ds(base, ch)], buf.at[0], sems[0]))

        for g in range(nchunk):
            bsel = g % 2
            if g + 1 < nchunk:
                copies.append(pltpu.async_copy(
                    codes_hbm.at[pl.ds(base + (g + 1) * ch, ch)],
                    buf.at[(g + 1) % 2], sems[(g + 1) % 2]))
            copies[g].wait()

            @plsc.parallel_loop(0, ch, step=128, unroll=4)
            def _(k, bsel=bsel):
                for u in range(8):
                    w = buf[bsel, pl.ds(k + u * 16, 16)]
                    a0 = (w << 4) & 0xFF0
                    a1 = lax.shift_right_logical(w, 4) & 0xFF0
                    a2 = lax.shift_right_logical(w, 12) & 0xFF0
                    a3 = lax.shift_right_logical(w, 20) & 0xFF0
                    plsc.addupdate_scatter(hist, [a0 + laneoff], ones16)
                    plsc.addupdate_scatter(hist, [a1 + laneoff], ones16)
                    plsc.addupdate_scatter(hist, [a2 + laneoff], ones16)
                    plsc.addupdate_scatter(hist, [a3 + laneoff], ones16)

        pltpu.sync_copy(hist, out_hbm.at[pl.ds(wid * hist_words, hist_words)])

    return sc_hist, nw


def kernel(embeddings, labels):
    b = embeddings.shape[0]
    labels = labels.astype(jnp.int32)

    en, stats = _tc_stats(embeddings, labels)

    nt = b // _TILE
    band_sizes = (4, 2, 2)
    assert sum(band_sizes) == nt
    acc = None
    base = 0
    for bt in band_sizes:
        codes = _tc_codes_band(en, labels, base, bt)
        sc_hist, nw = _make_sc_hist(bt * nt * _TILE * _TILE // 4)
        p = sc_hist(codes.reshape(-1))
        acc = p if acc is None else acc + p
        base += bt
    hist = jnp.sum(acc.reshape(nw, _NCODES, 16), axis=(0, 2))

    neg_hist = hist[:_NUM_STEPS]
    pos_hist = hist[_NUM_STEPS:2 * _NUM_STEPS]
    pos_hist = pos_hist / (jnp.sum(pos_hist) + 1e-16)
    neg_hist = neg_hist / (jnp.sum(neg_hist) + 1e-16)
    overlap = jnp.sum(jnp.minimum(pos_hist, neg_hist))

    s_all = stats[0, 0]
    s_eq = stats[0, 1]
    n_eq = stats[0, 2]
    trace = stats[0, 3]
    bf = jnp.float32(b)
    pos_mean = (s_eq - trace) / (n_eq - bf)
    neg_mean = (s_all - s_eq) / (bf * bf - n_eq)

    return overlap + jax.nn.relu(_MARGIN - (pos_mean - neg_mean))


# speedup vs baseline: 1.2459x; 1.0167x over previous
import functools

import jax
import jax.numpy as jnp
from jax import lax
from jax.experimental import pallas as pl
from jax.experimental.pallas import tpu as pltpu
from jax.experimental.pallas import tpu_sc as plsc

_NUM_STEPS = 100
_MARGIN = 0.1
_TILE = 512
_NCODES = 256


def _stats_body(emb_ref, labc_ref, norm_ref, stats_ref):
    e = emb_ref[...]
    n = e.shape[0]
    scale = 1.0 / jnp.maximum(jnp.sqrt(jnp.sum(e * e, axis=1, keepdims=True)),
                              1e-12)
    en = e * scale
    norm_ref[...] = en

    cls = lax.broadcasted_iota(jnp.int32, (128, n), 0)
    oh = (labc_ref[...] == cls).astype(jnp.float32)
    class_sums = lax.dot_general(oh, en, (((1,), (0,)), ((), ())),
                                 preferred_element_type=jnp.float32)
    counts = jnp.sum(oh, axis=1)

    s_eq = jnp.sum(class_sums * class_sums)
    n_eq = jnp.sum(counts * counts)
    colsum = jnp.sum(class_sums, axis=0)
    s_all = jnp.sum(colsum * colsum)
    trace = jnp.sum(en * en)

    row = lax.broadcasted_iota(jnp.int32, (8, 128), 0)
    lane = lax.broadcasted_iota(jnp.int32, (8, 128), 1)
    on_r0 = row == 0
    stats_ref[...] = (jnp.where(on_r0 & (lane == 0), s_all, 0.0)
                      + jnp.where(on_r0 & (lane == 1), s_eq, 0.0)
                      + jnp.where(on_r0 & (lane == 2), n_eq, 0.0)
                      + jnp.where(on_r0 & (lane == 3), trace, 0.0))


def _tc_stats(emb, labels):
    b, d = emb.shape
    return pl.pallas_call(
        _stats_body,
        out_shape=[
            jax.ShapeDtypeStruct((b, d), jnp.float32),
            jax.ShapeDtypeStruct((8, 128), jnp.float32),
        ],
    )(emb, labels.reshape(1, b))


def _codes_body(base_tile, enr_ref, enc_ref, labr_ref, labc_ref, codes_ref):
    bi = pl.program_id(0) + base_tile
    bj = pl.program_id(1)
    t = _TILE

    sim = lax.dot_general(enr_ref[...], enc_ref[...], (((1,), (1,)), ((), ())),
                          preferred_element_type=jnp.float32)

    eq = labr_ref[...] == labc_ref[...]
    half = 0.5 * _NUM_STEPS
    code = (jnp.minimum(sim * half, half - 0.5)
            + jnp.where(eq, 3.0 * half, half)).astype(jnp.int32)

    def pack4(c):
        q = t // 4
        w = (c[:q, :] | (c[q:2 * q, :] << 8)
             | (c[2 * q:3 * q, :] << 16) | (c[3 * q:, :] << 24))
        return w.reshape(codes_ref.shape)

    @pl.when(bi == bj)
    def _():
        ld = (lax.broadcasted_iota(jnp.int32, (t, t), 0)
              == lax.broadcasted_iota(jnp.int32, (t, t), 1))
        codes_ref[...] = pack4(jnp.where(ld, _NCODES - 1, code))

    @pl.when(bi != bj)
    def _():
        codes_ref[...] = pack4(code)


def _tc_codes_band(en, labels, base, bt):
    b, d = en.shape
    nt = b // _TILE
    rows_per_blk = _TILE * _TILE // 4 // 128
    labr = labels.reshape(b, 1)
    labc = labels.reshape(1, b)
    return pl.pallas_call(
        functools.partial(_codes_body, base),
        grid=(bt, nt),
        in_specs=[
            pl.BlockSpec((_TILE, d), lambda i, j, base=base: (base + i, 0)),
            pl.BlockSpec((_TILE, d), lambda i, j: (j, 0)),
            pl.BlockSpec((_TILE, 1), lambda i, j, base=base: (base + i, 0)),
            pl.BlockSpec((1, _TILE), lambda i, j: (0, j)),
        ],
        out_specs=pl.BlockSpec((rows_per_blk, 128),
                               lambda i, j, nt=nt: (i * nt + j, 0)),
        out_shape=jax.ShapeDtypeStruct((bt * nt * rows_per_blk, 128),
                                       jnp.int32),
    )(en, en, labr, labc)


def _make_sc_hist(total):
    info = plsc.get_sparse_core_info()
    nc, ns = info.num_cores, info.num_subcores
    nw = nc * ns
    words_per_w = total // nw
    ch = 32768
    nchunk = words_per_w // ch
    hist_words = 16 * _NCODES
    mesh = plsc.VectorSubcoreMesh(core_axis_name="c", subcore_axis_name="s")

    @functools.partial(
        pl.kernel, mesh=mesh,
        out_type=jax.ShapeDtypeStruct((nw * hist_words,), jnp.float32),
        compiler_params=pltpu.CompilerParams(needs_layout_passes=False),
        scratch_types=[
            pltpu.VMEM((2, ch), jnp.int32),
            pltpu.VMEM((hist_words,), jnp.float32),
            pltpu.SemaphoreType.DMA,
            pltpu.SemaphoreType.DMA,
        ],
    )
    def sc_hist(codes_hbm, out_hbm, buf, hist, sem0, sem1):
        wid = lax.axis_index("s") * nc + lax.axis_index("c")
        base = wid * words_per_w
        sems = (sem0, sem1)

        zero16 = jnp.zeros((16,), jnp.float32)

        def zinit(i, _):
            hist[pl.ds(i * 16, 16)] = zero16
            return 0

        lax.fori_loop(0, hist_words // 16, zinit, 0)

        ones16 = jnp.full((16,), 1.0, jnp.float32)
        laneoff = lax.iota(jnp.int32, 16)

        copies = []
        copies.append(pltpu.async_copy(
            codes_hbm.at[pl.ds(base, ch)], buf.at[0], sems[0]))

        for g in range(nchunk):
            bsel = g % 2
            if g + 1 < nchunk:
                copies.append(pltpu.async_copy(
                    codes_hbm.at[pl.ds(base + (g + 1) * ch, ch)],
                    buf.at[(g + 1) % 2], sems[(g + 1) % 2]))
            copies[g].wait()

            @plsc.parallel_loop(0, ch, step=128, unroll=4)
            def _(k, bsel=bsel):
                for u in range(8):
                    w = buf[bsel, pl.ds(k + u * 16, 16)]
                    a0 = (w << 4) & 0xFF0
                    a1 = lax.shift_right_logical(w, 4) & 0xFF0
                    a2 = lax.shift_right_logical(w, 12) & 0xFF0
                    a3 = lax.shift_right_logical(w, 20) & 0xFF0
                    plsc.addupdate_scatter(hist, [a0 + laneoff], ones16)
                    plsc.addupdate_scatter(hist, [a1 + laneoff], ones16)
                    plsc.addupdate_scatter(hist, [a2 + laneoff], ones16)
                    plsc.addupdate_scatter(hist, [a3 + laneoff], ones16)

        pltpu.sync_copy(hist, out_hbm.at[pl.ds(wid * hist_words, hist_words)])

    return sc_hist, nw


def kernel(embeddings, labels):
    b = embeddings.shape[0]
    labels = labels.astype(jnp.int32)

    en, stats = _tc_stats(embeddings, labels)

    nt = b // _TILE
    band_sizes = (4, 2, 2)
    assert sum(band_sizes) == nt
    acc = None
    base = 0
    for bt in band_sizes:
        codes = _tc_codes_band(en, labels, base, bt)
        sc_hist, nw = _make_sc_hist(bt * nt * _TILE * _TILE // 4)
        p = sc_hist(codes.reshape(-1))
        acc = p if acc is None else acc + p
        base += bt
    hist = jnp.sum(acc.reshape(nw, _NCODES, 16), axis=(0, 2))

    neg_hist = hist[:_NUM_STEPS]
    pos_hist = hist[_NUM_STEPS:2 * _NUM_STEPS]
    pos_hist = pos_hist / (jnp.sum(pos_hist) + 1e-16)
    neg_hist = neg_hist / (jnp.sum(neg_hist) + 1e-16)
    overlap = jnp.sum(jnp.minimum(pos_hist, neg_hist))

    s_all = stats[0, 0]
    s_eq = stats[0, 1]
    n_eq = stats[0, 2]
    trace = stats[0, 3]
    bf = jnp.float32(b)
    pos_mean = (s_eq - trace) / (n_eq - bf)
    neg_mean = (s_all - s_eq) / (bf * bf - n_eq)

    return overlap + jax.nn.relu(_MARGIN - (pos_mean - neg_mean))
